# baseline (speedup 1.0000x reference)
import functools

import jax
import jax.numpy as jnp
from jax import lax
from jax.experimental import pallas as pl
from jax.experimental.pallas import tpu as pltpu
from jax.experimental.pallas import tpu_sc as plsc

R = 16384
S = 4
RS = R // S
C = 1000
K = 200
NC, NS, L = 2, 16, 16
NW = NC * NS
RPW = RS // NW
B = 32
NB = RPW // B
FULL = K // L
LAST = K - L


@functools.partial(
    pl.kernel,
    mesh=plsc.VectorSubcoreMesh(core_axis_name="c", subcore_axis_name="s"),
    compiler_params=pltpu.CompilerParams(
        needs_layout_passes=False, use_tc_tiling_on_sc=True
    ),
    out_type=jax.ShapeDtypeStruct((RS, K), jnp.float32),
    scratch_types=[
        pltpu.VMEM((B, C), jnp.float32),
        pltpu.VMEM((B, K), jnp.int32),
        pltpu.VMEM((B, K), jnp.float32),
    ],
)
def _gather_kernel(x_hbm, ind_hbm, out_hbm, x_v, ind_v, out_v):
    wid = lax.axis_index("s") * NC + lax.axis_index("c")
    base_row = wid * RPW

    def do_block(b, _):
        row0 = base_row + b * B
        pltpu.sync_copy(x_hbm.at[pl.ds(row0, B)], x_v)
        pltpu.sync_copy(ind_hbm.at[pl.ds(row0, B)], ind_v)

        def do_row(r, _):
            row = jnp.full((L,), r, jnp.int32)
            for c in range(FULL):
                col = ind_v[r, pl.ds(c * L, L)]
                out_v[r, pl.ds(c * L, L)] = plsc.load_gather(x_v, [row, col])
            col = ind_v[r, pl.ds(LAST, L)]
            out_v[r, pl.ds(LAST, L)] = plsc.load_gather(x_v, [row, col])
            return 0

        lax.fori_loop(0, B, do_row, 0, unroll=False)
        pltpu.sync_copy(out_v, out_hbm.at[pl.ds(row0, B)])
        return 0

    lax.fori_loop(0, NB, do_block, 0, unroll=False)


def kernel(x, ind1):
    outs = [
        _gather_kernel(x[s * RS:(s + 1) * RS], ind1[s * RS:(s + 1) * RS])
        for s in range(S)
    ]
    return (jnp.concatenate(outs, axis=0),)


# speedup vs baseline: 1.1916x; 1.1916x over previous
import functools

import jax
import jax.numpy as jnp
from jax import lax
from jax.experimental import pallas as pl
from jax.experimental.pallas import tpu as pltpu
from jax.experimental.pallas import tpu_sc as plsc

R = 16384
C = 1000
K = 200
NC, NS, L = 2, 16, 16
NW = NC * NS
RPW = R // NW
B = 32
NB = RPW // B
FULL = K // L
LAST = K - L


@functools.partial(
    pl.kernel,
    mesh=plsc.VectorSubcoreMesh(core_axis_name="c", subcore_axis_name="s"),
    compiler_params=pltpu.CompilerParams(needs_layout_passes=False),
    out_type=jax.ShapeDtypeStruct((R, K), jnp.float32),
    scratch_types=[
        pltpu.VMEM((2, B, C), jnp.float32),
        pltpu.VMEM((2, B, K), jnp.int32),
        pltpu.VMEM((2, B, K), jnp.float32),
        pltpu.SemaphoreType.DMA((2,)),
        pltpu.SemaphoreType.DMA((2,)),
        pltpu.SemaphoreType.DMA((2,)),
    ],
)
def _gather_kernel(x_hbm, ind_hbm, out_hbm, x_v, ind_v, out_v, sx, si, so):
    wid = lax.axis_index("s") * NC + lax.axis_index("c")
    base_row = wid * RPW

    def in_copies(b, buf):
        row0 = base_row + b * B
        cx = pltpu.make_async_copy(x_hbm.at[pl.ds(row0, B)], x_v.at[buf], sx.at[buf])
        ci = pltpu.make_async_copy(ind_hbm.at[pl.ds(row0, B)], ind_v.at[buf], si.at[buf])
        return cx, ci

    def out_copy(b, buf):
        row0 = base_row + b * B
        return pltpu.make_async_copy(out_v.at[buf], out_hbm.at[pl.ds(row0, B)], so.at[buf])

    cx0, ci0 = in_copies(0, 0)
    cx0.start()
    ci0.start()

    def do_block(b, _):
        cur = lax.rem(b, 2)
        nxt = 1 - cur

        @pl.when(b + 1 < NB)
        def _():
            cx, ci = in_copies(b + 1, nxt)
            cx.start()
            ci.start()

        cxc, cic = in_copies(b, cur)
        cxc.wait()
        cic.wait()

        @pl.when(b >= 2)
        def _():
            out_copy(b - 2, cur).wait()

        def do_row(r, _):
            row = jnp.full((L,), r, jnp.int32)
            for c in range(FULL):
                col = ind_v[cur, r, pl.ds(c * L, L)]
                out_v[cur, r, pl.ds(c * L, L)] = plsc.load_gather(
                    x_v.at[cur], [row, col]
                )
            col = ind_v[cur, r, pl.ds(LAST, L)]
            out_v[cur, r, pl.ds(LAST, L)] = plsc.load_gather(x_v.at[cur], [row, col])
            return 0

        lax.fori_loop(0, B, do_row, 0, unroll=False)
        out_copy(b, cur).start()
        return 0

    lax.fori_loop(0, NB, do_block, 0, unroll=False)

    out_copy(NB - 2, 0).wait()
    out_copy(NB - 1, 1).wait()


def kernel(x, ind1):
    return (_gather_kernel(x, ind1),)


# speedup vs baseline: 1.5634x; 1.3120x over previous
import functools

import jax
import jax.numpy as jnp
from jax import lax
from jax.experimental import pallas as pl
from jax.experimental.pallas import tpu as pltpu
from jax.experimental.pallas import tpu_sc as plsc

R = 16384
C = 1000
K = 200
NC, NS, L = 2, 16, 16
NW = NC * NS
RPW = R // NW
B = 32
NB = RPW // B
FULL = K // L
LAST = K - L


@functools.partial(
    pl.kernel,
    mesh=plsc.VectorSubcoreMesh(core_axis_name="c", subcore_axis_name="s"),
    compiler_params=pltpu.CompilerParams(needs_layout_passes=False),
    out_type=jax.ShapeDtypeStruct((R, K), jnp.float32),
    scratch_types=[
        pltpu.VMEM((2, B, C), jnp.float32),
        pltpu.VMEM((2, B, K), jnp.int32),
        pltpu.VMEM((2, B, K), jnp.float32),
        pltpu.SemaphoreType.DMA((2,)),
        pltpu.SemaphoreType.DMA((2,)),
        pltpu.SemaphoreType.DMA((2,)),
    ],
)
def _gather_kernel(x_hbm, ind_hbm, out_hbm, x_v, ind_v, out_v, sx, si, so):
    wid = lax.axis_index("s") * NC + lax.axis_index("c")
    base_row = wid * RPW

    def in_copies(b, buf):
        row0 = base_row + b * B
        cx = pltpu.make_async_copy(x_hbm.at[pl.ds(row0, B)], x_v.at[buf], sx.at[buf])
        ci = pltpu.make_async_copy(ind_hbm.at[pl.ds(row0, B)], ind_v.at[buf], si.at[buf])
        return cx, ci

    def out_copy(b, buf):
        row0 = base_row + b * B
        return pltpu.make_async_copy(out_v.at[buf], out_hbm.at[pl.ds(row0, B)], so.at[buf])

    def compute_block(b, buf):
        def do_row(r, _):
            row = jnp.full((L,), r, jnp.int32)
            for c in range(FULL):
                col = ind_v[buf, r, pl.ds(c * L, L)]
                out_v[buf, r, pl.ds(c * L, L)] = plsc.load_gather(
                    x_v.at[buf], [row, col]
                )
            col = ind_v[buf, r, pl.ds(LAST, L)]
            out_v[buf, r, pl.ds(LAST, L)] = plsc.load_gather(x_v.at[buf], [row, col])
            return 0

        lax.fori_loop(0, B, do_row, 0, unroll=2)

    def start_in(b, buf):
        cx, ci = in_copies(b, buf)
        cx.start()
        ci.start()

    def wait_in(b, buf):
        cx, ci = in_copies(b, buf)
        cx.wait()
        ci.wait()

    start_in(0, 0)
    start_in(1, 1)

    def do_pair(i, _):
        b0 = 2 * i
        b1 = b0 + 1

        wait_in(b0, 0)

        @pl.when(i >= 1)
        def _():
            out_copy(b0 - 2, 0).wait()

        compute_block(b0, 0)
        out_copy(b0, 0).start()

        @pl.when(b0 + 2 < NB)
        def _():
            start_in(b0 + 2, 0)

        wait_in(b1, 1)

        @pl.when(i >= 1)
        def _():
            out_copy(b1 - 2, 1).wait()

        compute_block(b1, 1)
        out_copy(b1, 1).start()

        @pl.when(b1 + 2 < NB)
        def _():
            start_in(b1 + 2, 1)

        return 0

    lax.fori_loop(0, NB // 2, do_pair, 0, unroll=False)

    out_copy(NB - 2, 0).wait()
    out_copy(NB - 1, 1).wait()


def kernel(x, ind1):
    return (_gather_kernel(x, ind1),)


# speedup vs baseline: 1.6916x; 1.0820x over previous
import functools

import jax
import jax.numpy as jnp
from jax import lax
from jax.experimental import pallas as pl
from jax.experimental.pallas import tpu as pltpu
from jax.experimental.pallas import tpu_sc as plsc

R = 16384
C = 1000
K = 200
NC, NS, L = 2, 16, 16
NW = NC * NS
CPW = R // NW
W = 128
NST = CPW // W
JT = K // 8


@functools.partial(
    pl.kernel,
    mesh=plsc.VectorSubcoreMesh(core_axis_name="c", subcore_axis_name="s"),
    compiler_params=pltpu.CompilerParams(needs_layout_passes=False),
    out_type=jax.ShapeDtypeStruct((K, R), jnp.float32),
    scratch_types=[
        pltpu.VMEM((C, W), jnp.float32),
        pltpu.VMEM((8, W), jnp.float32),
        pltpu.VMEM((8, W), jnp.float32),
        pltpu.SemaphoreType.DMA((2,)),
        pltpu.SemaphoreType.DMA((2,)),
    ],
)
def _gather_kernel(xt_hbm, it_hbm, ot_hbm, slab, tb0, tb1, si, so):
    wid = lax.axis_index("s") * NC + lax.axis_index("c")
    base_col = wid * CPW

    cvecs = [
        lax.iota(jnp.int32, L) + ch * L for ch in range(W // L)
    ]

    def do_stripe(st, _):
        c0 = base_col + st * W

        def ind_cp(jt, buf, sem):
            return pltpu.make_async_copy(
                it_hbm.at[pl.ds(jt * 8, 8), pl.ds(c0, W)], buf, sem
            )

        def out_cp(jt, buf, sem):
            return pltpu.make_async_copy(
                buf, ot_hbm.at[pl.ds(jt * 8, 8), pl.ds(c0, W)], sem
            )

        pltpu.sync_copy(xt_hbm.at[:, pl.ds(c0, W)], slab)

        @pl.when(st >= 1)
        def _():
            out_cp(JT - 1, tb0, so.at[0]).wait()

        ind_cp(0, tb0, si.at[0]).start()

        def do_group(jt, buf, bo, ib, ibo):
            ind_cp(jt, buf, si.at[ib]).wait()
            for jj in range(8):
                for ch in range(W // L):
                    tf = buf[jj, pl.ds(ch * L, L)]
                    t = plsc.bitcast(tf, jnp.int32)
                    val = plsc.load_gather(slab, [t, cvecs[ch]])
                    buf[jj, pl.ds(ch * L, L)] = val

            @pl.when(jt >= 1)
            def _():
                out_cp(jt - 1, bo, so.at[ibo]).wait()

            @pl.when(jt + 1 < JT)
            def _():
                ind_cp(jt + 1, bo, si.at[ibo]).start()

            out_cp(jt, buf, so.at[ib]).start()

        do_group(0, tb0, tb1, 0, 1)

        def do_pair(i, _):
            do_group(2 * i + 1, tb1, tb0, 1, 0)
            do_group(2 * i + 2, tb0, tb1, 0, 1)
            return 0

        lax.fori_loop(0, (JT - 1) // 2, do_pair, 0, unroll=False)
        return 0

    lax.fori_loop(0, NST, do_stripe, 0, unroll=False)

    pltpu.make_async_copy(
        tb0, ot_hbm.at[pl.ds((JT - 1) * 8, 8), pl.ds(base_col + (NST - 1) * W, W)],
        so.at[0],
    ).wait()


def kernel(x, ind1):
    xt = x.T
    it = lax.bitcast_convert_type(ind1, jnp.float32).T
    ot = _gather_kernel(xt, it)
    return (ot.T,)


# speedup vs baseline: 1.8918x; 1.1183x over previous
import functools

import jax
import jax.numpy as jnp
from jax import lax
from jax.experimental import pallas as pl
from jax.experimental.pallas import tpu as pltpu
from jax.experimental.pallas import tpu_sc as plsc

R = 16384
C = 1000
K = 200
NC, NS, L = 2, 16, 16
NW = NC * NS
CPW = R // NW
W = 128
NST = CPW // W
JT = K // 8


@functools.partial(
    pl.kernel,
    mesh=plsc.VectorSubcoreMesh(core_axis_name="c", subcore_axis_name="s"),
    compiler_params=pltpu.CompilerParams(needs_layout_passes=False),
    out_type=jax.ShapeDtypeStruct((K, R), jnp.float32),
    scratch_types=[
        pltpu.VMEM((C, W), jnp.float32),
        pltpu.VMEM((8, W), jnp.float32),
        pltpu.VMEM((8, W), jnp.float32),
        pltpu.SemaphoreType.DMA((2,)),
        pltpu.SemaphoreType.DMA((2,)),
    ],
)
def _gather_kernel(xt_hbm, it_hbm, ot_hbm, slab, tb0, tb1, si, so):
    wid = lax.axis_index("s") * NC + lax.axis_index("c")
    base_col = wid * CPW

    cvecs = [
        lax.iota(jnp.int32, L) + ch * L for ch in range(W // L)
    ]

    def do_stripe(st, _):
        c0 = base_col + st * W

        def ind_cp(jt, buf, sem):
            return pltpu.make_async_copy(
                it_hbm.at[pl.ds(jt * 8, 8), pl.ds(c0, W)], buf, sem
            )

        def out_cp(jt, buf, sem):
            return pltpu.make_async_copy(
                buf, ot_hbm.at[pl.ds(jt * 8, 8), pl.ds(c0, W)], sem
            )

        pltpu.sync_copy(xt_hbm.at[:, pl.ds(c0, W)], slab)

        @pl.when(st >= 1)
        def _():
            out_cp(JT - 1, tb0, so.at[0]).wait()

        ind_cp(0, tb0, si.at[0]).start()

        def do_group(jt, buf, bo, ib, ibo):
            ind_cp(jt, buf, si.at[ib]).wait()
            for jj in range(8):
                ts = [
                    plsc.bitcast(buf[jj, pl.ds(ch * L, L)], jnp.int32)
                    for ch in range(W // L)
                ]
                vals = [
                    plsc.load_gather(slab, [ts[ch], cvecs[ch]])
                    for ch in range(W // L)
                ]
                for ch in range(W // L):
                    buf[jj, pl.ds(ch * L, L)] = vals[ch]

            @pl.when(jt >= 1)
            def _():
                out_cp(jt - 1, bo, so.at[ibo]).wait()

            @pl.when(jt + 1 < JT)
            def _():
                ind_cp(jt + 1, bo, si.at[ibo]).start()

            out_cp(jt, buf, so.at[ib]).start()

        do_group(0, tb0, tb1, 0, 1)

        def do_pair(i, _):
            do_group(2 * i + 1, tb1, tb0, 1, 0)
            do_group(2 * i + 2, tb0, tb1, 0, 1)
            return 0

        lax.fori_loop(0, (JT - 1) // 2, do_pair, 0, unroll=False)
        return 0

    lax.fori_loop(0, NST, do_stripe, 0, unroll=False)

    pltpu.make_async_copy(
        tb0, ot_hbm.at[pl.ds((JT - 1) * 8, 8), pl.ds(base_col + (NST - 1) * W, W)],
        so.at[0],
    ).wait()


def kernel(x, ind1):
    xt = x.T
    it = lax.bitcast_convert_type(ind1, jnp.float32).T
    ot = _gather_kernel(xt, it)
    return (ot.T,)
